# baseline (speedup 1.0000x reference)
import jax
import jax.numpy as jnp
from jax.experimental import pallas as pl
from jax.experimental.pallas import tpu as pltpu

EMB = 256
CODES = 1024
PIX = 1024
BATCH = 16
COMMIT = 0.25


def _vq_kernel(x_ref, e_ref, out_ref, oh_ref, loss_ref, et_ref):
    n = pl.program_id(0)

    @pl.when(n == 0)
    def _():
        et_ref[...] = e_ref[...].T
        loss_ref[...] = jnp.zeros_like(loss_ref)

    x = x_ref[0].reshape(EMB, PIX)
    et = et_ref[...]
    scores_t = jax.lax.dot_general(
        et, x, (((1,), (0,)), ((), ())), preferred_element_type=jnp.float32
    )
    e_norm = jnp.sum(et * et, axis=1, keepdims=True)
    d = e_norm - 2.0 * scores_t
    dmin = jnp.min(d, axis=0, keepdims=True)
    iota_c = jax.lax.broadcasted_iota(jnp.int32, (CODES, PIX), 0)
    idx = jnp.min(jnp.where(d == dmin, iota_c, CODES), axis=0, keepdims=True)
    oh_t = jnp.where(iota_c == idx, 1.0, 0.0)
    q = jax.lax.dot_general(
        e_ref[...], oh_t, (((1,), (0,)), ((), ())),
        preferred_element_type=jnp.float32,
    )
    out_ref[0] = q.reshape(EMB, 32, 32)
    idx_col = idx.T
    iota_p = jax.lax.broadcasted_iota(jnp.int32, (PIX, CODES), 1)
    oh_ref[...] = jnp.where(iota_p == idx_col, 1.0, 0.0)
    z_norm = jnp.sum(x * x, axis=0, keepdims=True)
    loss_ref[...] += jnp.sum(z_norm) + jnp.sum(dmin)


def kernel(inputs, embed):
    out, onehot, loss_sum = pl.pallas_call(
        _vq_kernel,
        grid=(BATCH,),
        in_specs=[
            pl.BlockSpec((1, EMB, 32, 32), lambda n: (n, 0, 0, 0)),
            pl.BlockSpec((EMB, CODES), lambda n: (0, 0)),
        ],
        out_specs=[
            pl.BlockSpec((1, EMB, 32, 32), lambda n: (n, 0, 0, 0)),
            pl.BlockSpec((PIX, CODES), lambda n: (n, 0)),
            pl.BlockSpec((1, 1), lambda n: (0, 0)),
        ],
        out_shape=[
            jax.ShapeDtypeStruct((BATCH, EMB, 32, 32), jnp.float32),
            jax.ShapeDtypeStruct((BATCH * PIX, CODES), jnp.float32),
            jax.ShapeDtypeStruct((1, 1), jnp.float32),
        ],
        scratch_shapes=[pltpu.VMEM((CODES, EMB), jnp.float32)],
        compiler_params=pltpu.CompilerParams(
            dimension_semantics=("arbitrary",),
        ),
    )(inputs, embed)
    loss = loss_sum[0, 0] * ((1.0 + COMMIT) / (BATCH * PIX * EMB))
    return (out, loss, onehot)


# speedup vs baseline: 3.3362x; 3.3362x over previous
import jax
import jax.numpy as jnp
from jax.experimental import pallas as pl
from jax.experimental.pallas import tpu as pltpu

EMB = 256
CODES = 1024
PIX = 1024
BATCH = 16
COMMIT = 0.25


def _vq_kernel(x_ref, e_ref, out_ref, oh_ref, loss_ref, et_ref, en_ref):
    n = pl.program_id(0)

    @pl.when(n == 0)
    def _():
        e = e_ref[...]
        et_ref[...] = e.T
        en_ref[...] = jnp.sum(e * e, axis=0, keepdims=True)
        loss_ref[...] = jnp.zeros_like(loss_ref)

    x = x_ref[...]
    scores = jax.lax.dot_general(
        x, e_ref[...], (((1,), (0,)), ((), ())),
        preferred_element_type=jnp.float32,
    )
    d = en_ref[...] - 2.0 * scores
    dmin = jnp.min(d, axis=1, keepdims=True)
    iota_c = jax.lax.broadcasted_iota(jnp.int32, (PIX, CODES), 1)
    idx = jnp.min(jnp.where(d == dmin, iota_c, CODES), axis=1, keepdims=True)
    oh = jnp.where(iota_c == idx, 1.0, 0.0)
    oh_ref[...] = oh
    out_ref[...] = jax.lax.dot_general(
        oh, et_ref[...], (((1,), (0,)), ((), ())),
        preferred_element_type=jnp.float32,
    )
    z_norm = jnp.sum(x * x, axis=1, keepdims=True)
    loss_ref[...] += jnp.sum(z_norm) + jnp.sum(dmin)


def kernel(inputs, embed):
    flat = jnp.transpose(inputs, (0, 2, 3, 1)).reshape(BATCH * PIX, EMB)
    quant, onehot, loss_sum = pl.pallas_call(
        _vq_kernel,
        grid=(BATCH,),
        in_specs=[
            pl.BlockSpec((PIX, EMB), lambda n: (n, 0)),
            pl.BlockSpec((EMB, CODES), lambda n: (0, 0)),
        ],
        out_specs=[
            pl.BlockSpec((PIX, EMB), lambda n: (n, 0)),
            pl.BlockSpec((PIX, CODES), lambda n: (n, 0)),
            pl.BlockSpec((1, 1), lambda n: (0, 0)),
        ],
        out_shape=[
            jax.ShapeDtypeStruct((BATCH * PIX, EMB), jnp.float32),
            jax.ShapeDtypeStruct((BATCH * PIX, CODES), jnp.float32),
            jax.ShapeDtypeStruct((1, 1), jnp.float32),
        ],
        scratch_shapes=[
            pltpu.VMEM((CODES, EMB), jnp.float32),
            pltpu.VMEM((1, CODES), jnp.float32),
        ],
        compiler_params=pltpu.CompilerParams(
            dimension_semantics=("arbitrary",),
        ),
    )(flat, embed)
    loss = loss_sum[0, 0] * ((1.0 + COMMIT) / (BATCH * PIX * EMB))
    out = jnp.transpose(quant.reshape(BATCH, 32, 32, EMB), (0, 3, 1, 2))
    return (out, loss, onehot)
